# baseline (speedup 1.0000x reference)
import functools

import jax
import jax.numpy as jnp
from jax import lax
from jax.experimental import pallas as pl
from jax.experimental.pallas import tpu as pltpu

N_JRNL = 1000000
JRNL_DIM = 32
MESH_SIZE = 28340
HIDDEN_SIZE = 128
N_PROBES = 3
BATCH = 1024
CTX_DIM = HIDDEN_SIZE * N_PROBES
IN_FEAT = CTX_DIM + JRNL_DIM

_BLK_N = 2048

_CPS = 16
_GSTEPS = BATCH // _CPS
_LANES = 128


def _gather_body(c_ref, *refs):
    l_ref = refs[_CPS]
    out_ref = refs[_CPS + 1]
    g = pl.program_id(0)
    del g, c_ref
    chunks = jnp.concatenate([refs[k][...] for k in range(_CPS)], axis=1)
    lane = lax.broadcasted_iota(jnp.int32, (_CPS, _CPS * _LANES), 1)
    row = lax.broadcasted_iota(jnp.int32, (_CPS, _CPS * _LANES), 0)
    sel = ((lane % _LANES) == l_ref[0]) & ((lane // _LANES) == row)
    st = sel.astype(jnp.float32)
    cols = lax.dot_general(
        chunks, st, (((1,), (1,)), ((), ())),
        preferred_element_type=jnp.float32,
    )
    out_ref[0] = cols


def _chunk_spec(k):
    return pl.BlockSpec(
        (JRNL_DIM, _LANES), lambda g, c_ref, k=k: (0, c_ref[_CPS * g + k]))


def _mm_body(wt_ref, ctx_ref, emb_ref, b_ref, out_ref, comb_ref):
    @pl.when(pl.program_id(0) == 0)
    def _init():
        comb_ref[0:CTX_DIM, :] = jnp.transpose(ctx_ref[...])
        for g in range(_GSTEPS):
            comb_ref[CTX_DIM:IN_FEAT, _CPS * g:_CPS * (g + 1)] = emb_ref[g]

    acc = lax.dot_general(
        wt_ref[...], comb_ref[...],
        (((0,), (0,)), ((), ())),
        preferred_element_type=jnp.float32,
    )
    out_ref[...] = acc + b_ref[...]


@jax.jit
def kernel(jrnl_variable, context_vectors, emb_table, W, b):
    idx = jrnl_variable.reshape((BATCH,))
    c = idx // _LANES
    l2d = (idx % _LANES).reshape((_GSTEPS, _CPS, 1))
    table_t = emb_table.T
    wt = W.T
    b2d = b.reshape((MESH_SIZE, 1))

    emb_cols = pl.pallas_call(
        _gather_body,
        grid_spec=pltpu.PrefetchScalarGridSpec(
            num_scalar_prefetch=1,
            grid=(_GSTEPS,),
            in_specs=[_chunk_spec(k) for k in range(_CPS)] + [
                pl.BlockSpec((1, _CPS, 1), lambda g, c_ref: (g, 0, 0)),
            ],
            out_specs=pl.BlockSpec(
                (1, JRNL_DIM, _CPS), lambda g, c_ref: (g, 0, 0)),
        ),
        out_shape=jax.ShapeDtypeStruct((_GSTEPS, JRNL_DIM, _CPS),
                                       jnp.float32),
        compiler_params=pltpu.CompilerParams(
            dimension_semantics=("arbitrary",),
        ),
    )(c, *([table_t] * _CPS), l2d)

    n_blocks = pl.cdiv(MESH_SIZE, _BLK_N)
    out_t = pl.pallas_call(
        _mm_body,
        grid=(n_blocks,),
        in_specs=[
            pl.BlockSpec((IN_FEAT, _BLK_N), lambda i: (0, i)),
            pl.BlockSpec((BATCH, CTX_DIM), lambda i: (0, 0)),
            pl.BlockSpec((_GSTEPS, JRNL_DIM, _CPS), lambda i: (0, 0, 0)),
            pl.BlockSpec((_BLK_N, 1), lambda i: (i, 0)),
        ],
        out_specs=pl.BlockSpec((_BLK_N, BATCH), lambda i: (i, 0)),
        out_shape=jax.ShapeDtypeStruct((MESH_SIZE, BATCH), jnp.float32),
        scratch_shapes=[pltpu.VMEM((IN_FEAT, BATCH), jnp.float32)],
        compiler_params=pltpu.CompilerParams(
            dimension_semantics=("arbitrary",),
        ),
    )(wt, context_vectors, emb_cols, b2d)
    return out_t.T


# speedup vs baseline: 1.2911x; 1.2911x over previous
import functools

import jax
import jax.numpy as jnp
from jax import lax
from jax.experimental import pallas as pl
from jax.experimental.pallas import tpu as pltpu

N_JRNL = 1000000
JRNL_DIM = 32
MESH_SIZE = 28340
HIDDEN_SIZE = 128
N_PROBES = 3
BATCH = 1024
CTX_DIM = HIDDEN_SIZE * N_PROBES
IN_FEAT = CTX_DIM + JRNL_DIM

_BLK_N = 2048

_GRP = 16
_NGRP = BATCH // _GRP
_LANES = 128


def _gather_body(c_ref, tbl_ref, l_ref, out_ref, g_ref, sem):
    def _issue(j, _):
        c = c_ref[j]
        pltpu.make_async_copy(
            tbl_ref.at[:, pl.ds(c * _LANES, _LANES)], g_ref.at[j], sem,
        ).start()
        return 0

    lax.fori_loop(0, BATCH, _issue, 0)

    def _drain(j, _):
        pltpu.make_async_copy(
            tbl_ref.at[:, pl.ds(0, _LANES)], g_ref.at[0], sem,
        ).wait()
        return 0

    lax.fori_loop(0, BATCH, _drain, 0)

    lane = lax.broadcasted_iota(jnp.int32, (_GRP, _GRP * _LANES), 1)
    row = lax.broadcasted_iota(jnp.int32, (_GRP, _GRP * _LANES), 0)
    for gq in range(_NGRP):
        chunks = jnp.concatenate(
            [g_ref[_GRP * gq + k] for k in range(_GRP)], axis=1)
        sel = ((lane % _LANES) == l_ref[gq]) & ((lane // _LANES) == row)
        cols = lax.dot_general(
            chunks, sel.astype(jnp.float32), (((1,), (1,)), ((), ())),
            preferred_element_type=jnp.float32,
        )
        out_ref[:, _GRP * gq:_GRP * (gq + 1)] = cols


def _mm_body(wt_ref, ctx_ref, emb_ref, b_ref, out_ref, comb_ref):
    @pl.when(pl.program_id(0) == 0)
    def _init():
        comb_ref[0:CTX_DIM, :] = jnp.transpose(ctx_ref[...])
        comb_ref[CTX_DIM:IN_FEAT, :] = emb_ref[...]

    acc = lax.dot_general(
        wt_ref[...], comb_ref[...],
        (((0,), (0,)), ((), ())),
        preferred_element_type=jnp.float32,
    )
    out_ref[...] = acc + b_ref[...]


@jax.jit
def kernel(jrnl_variable, context_vectors, emb_table, W, b):
    idx = jrnl_variable.reshape((BATCH,))
    c = idx // _LANES
    l3 = (idx % _LANES).reshape((_NGRP, _GRP, 1))
    table_t = emb_table.T
    wt = W.T
    b2d = b.reshape((MESH_SIZE, 1))

    emb_t = pl.pallas_call(
        _gather_body,
        grid_spec=pltpu.PrefetchScalarGridSpec(
            num_scalar_prefetch=1,
            grid=(1,),
            in_specs=[
                pl.BlockSpec(memory_space=pl.ANY),
                pl.BlockSpec((_NGRP, _GRP, 1), lambda g, c_ref: (0, 0, 0)),
            ],
            out_specs=pl.BlockSpec((JRNL_DIM, BATCH), lambda g, c_ref: (0, 0)),
            scratch_shapes=[
                pltpu.VMEM((BATCH, JRNL_DIM, _LANES), jnp.float32),
                pltpu.SemaphoreType.DMA,
            ],
        ),
        out_shape=jax.ShapeDtypeStruct((JRNL_DIM, BATCH), jnp.float32),
        compiler_params=pltpu.CompilerParams(
            dimension_semantics=("arbitrary",),
        ),
    )(c, table_t, l3)

    n_blocks = pl.cdiv(MESH_SIZE, _BLK_N)
    out_t = pl.pallas_call(
        _mm_body,
        grid=(n_blocks,),
        in_specs=[
            pl.BlockSpec((IN_FEAT, _BLK_N), lambda i: (0, i)),
            pl.BlockSpec((BATCH, CTX_DIM), lambda i: (0, 0)),
            pl.BlockSpec((JRNL_DIM, BATCH), lambda i: (0, 0)),
            pl.BlockSpec((_BLK_N, 1), lambda i: (i, 0)),
        ],
        out_specs=pl.BlockSpec((_BLK_N, BATCH), lambda i: (i, 0)),
        out_shape=jax.ShapeDtypeStruct((MESH_SIZE, BATCH), jnp.float32),
        scratch_shapes=[pltpu.VMEM((IN_FEAT, BATCH), jnp.float32)],
        compiler_params=pltpu.CompilerParams(
            dimension_semantics=("arbitrary",),
        ),
    )(wt, context_vectors, emb_t, b2d)
    return out_t.T


# speedup vs baseline: 1.3355x; 1.0343x over previous
import functools

import jax
import jax.numpy as jnp
from jax import lax
from jax.experimental import pallas as pl
from jax.experimental.pallas import tpu as pltpu

N_JRNL = 1000000
JRNL_DIM = 32
MESH_SIZE = 28340
HIDDEN_SIZE = 128
N_PROBES = 3
BATCH = 1024
CTX_DIM = HIDDEN_SIZE * N_PROBES
IN_FEAT = CTX_DIM + JRNL_DIM

_BLK_N = 2048

_GRP = 16
_NGRP = BATCH // _GRP
_LANES = 128


def _gather_body(c_ref, tbl_ref, l_ref, out_ref, g_ref, sems):
    def _issue(j, _):
        c = c_ref[j]
        pltpu.make_async_copy(
            tbl_ref.at[:, pl.ds(c * _LANES, _LANES)], g_ref.at[j],
            sems.at[j // _GRP],
        ).start()
        return 0

    lax.fori_loop(0, BATCH, _issue, 0, unroll=8)

    lane = lax.broadcasted_iota(jnp.int32, (_GRP, _GRP * _LANES), 1)
    row = lax.broadcasted_iota(jnp.int32, (_GRP, _GRP * _LANES), 0)
    for gq in range(_NGRP):
        grp = g_ref.at[pl.ds(_GRP * gq, _GRP)]
        pltpu.make_async_copy(grp, grp, sems.at[gq]).wait()
        chunks = jnp.concatenate(
            [g_ref[_GRP * gq + k] for k in range(_GRP)], axis=1)
        sel = ((lane % _LANES) == l_ref[gq]) & ((lane // _LANES) == row)
        cols = lax.dot_general(
            chunks, sel.astype(jnp.float32), (((1,), (1,)), ((), ())),
            preferred_element_type=jnp.float32,
        )
        out_ref[:, _GRP * gq:_GRP * (gq + 1)] = cols


def _mm_body(wt_ref, ctx_ref, emb_ref, b_ref, out_ref, comb_ref):
    @pl.when(pl.program_id(0) == 0)
    def _init():
        comb_ref[0:CTX_DIM, :] = jnp.transpose(ctx_ref[...])
        comb_ref[CTX_DIM:IN_FEAT, :] = emb_ref[...]

    acc = lax.dot_general(
        wt_ref[...], comb_ref[...],
        (((0,), (0,)), ((), ())),
        preferred_element_type=jnp.float32,
    )
    out_ref[...] = acc + jnp.transpose(b_ref[...])


@jax.jit
def kernel(jrnl_variable, context_vectors, emb_table, W, b):
    idx = jrnl_variable.reshape((BATCH,))
    c = idx // _LANES
    l3 = (idx % _LANES).reshape((_NGRP, _GRP, 1))
    table_t = emb_table.T
    wt = W.T
    b2d = b.reshape((1, MESH_SIZE))

    emb_t = pl.pallas_call(
        _gather_body,
        grid_spec=pltpu.PrefetchScalarGridSpec(
            num_scalar_prefetch=1,
            grid=(1,),
            in_specs=[
                pl.BlockSpec(memory_space=pl.ANY),
                pl.BlockSpec((_NGRP, _GRP, 1), lambda g, c_ref: (0, 0, 0)),
            ],
            out_specs=pl.BlockSpec((JRNL_DIM, BATCH), lambda g, c_ref: (0, 0)),
            scratch_shapes=[
                pltpu.VMEM((BATCH, JRNL_DIM, _LANES), jnp.float32),
                pltpu.SemaphoreType.DMA((_NGRP,)),
            ],
        ),
        out_shape=jax.ShapeDtypeStruct((JRNL_DIM, BATCH), jnp.float32),
        compiler_params=pltpu.CompilerParams(
            dimension_semantics=("arbitrary",),
        ),
    )(c, table_t, l3)

    n_blocks = pl.cdiv(MESH_SIZE, _BLK_N)
    out_t = pl.pallas_call(
        _mm_body,
        grid=(n_blocks,),
        in_specs=[
            pl.BlockSpec((IN_FEAT, _BLK_N), lambda i: (0, i)),
            pl.BlockSpec((BATCH, CTX_DIM), lambda i: (0, 0)),
            pl.BlockSpec((JRNL_DIM, BATCH), lambda i: (0, 0)),
            pl.BlockSpec((1, _BLK_N), lambda i: (0, i)),
        ],
        out_specs=pl.BlockSpec((_BLK_N, BATCH), lambda i: (i, 0)),
        out_shape=jax.ShapeDtypeStruct((MESH_SIZE, BATCH), jnp.float32),
        scratch_shapes=[pltpu.VMEM((IN_FEAT, BATCH), jnp.float32)],
        compiler_params=pltpu.CompilerParams(
            dimension_semantics=("arbitrary",),
        ),
    )(wt, context_vectors, emb_t, b2d)
    return out_t.T


# speedup vs baseline: 1.6790x; 1.2573x over previous
import functools

import jax
import jax.numpy as jnp
from jax import lax
from jax.experimental import pallas as pl
from jax.experimental.pallas import tpu as pltpu

N_JRNL = 1000000
JRNL_DIM = 32
MESH_SIZE = 28340
HIDDEN_SIZE = 128
N_PROBES = 3
BATCH = 1024
CTX_DIM = HIDDEN_SIZE * N_PROBES
IN_FEAT = CTX_DIM + JRNL_DIM

_BLK_N = 2048

_GRP = 16
_NGRP = BATCH // _GRP
_LANES = 128


def _gather_body(c_ref, tbl_ref, l_ref, out_ref, g_ref, sem):
    def _issue(j, _):
        c = c_ref[j]
        pltpu.make_async_copy(
            tbl_ref.at[:, pl.ds(c * _LANES, _LANES)], g_ref.at[j], sem,
        ).start()
        return 0

    lax.fori_loop(0, BATCH, _issue, 0, unroll=8)

    for gq in range(_NGRP):
        grp = g_ref.at[pl.ds(_GRP * gq, _GRP)]
        pltpu.make_async_copy(grp, grp, sem).wait()

    lane = lax.broadcasted_iota(jnp.int32, (_GRP, _GRP * _LANES), 1)
    row = lax.broadcasted_iota(jnp.int32, (_GRP, _GRP * _LANES), 0)
    for gq in range(_NGRP):
        chunks = jnp.concatenate(
            [g_ref[_GRP * gq + k] for k in range(_GRP)], axis=1)
        sel = ((lane % _LANES) == l_ref[gq]) & ((lane // _LANES) == row)
        cols = lax.dot_general(
            chunks, sel.astype(jnp.float32), (((1,), (1,)), ((), ())),
            preferred_element_type=jnp.float32,
        )
        out_ref[:, _GRP * gq:_GRP * (gq + 1)] = cols


def _mm_body(wt_ref, ctx_ref, emb_ref, b_ref, out_ref, comb_ref):
    @pl.when(pl.program_id(0) == 0)
    def _init():
        comb_ref[0:CTX_DIM, :] = jnp.transpose(ctx_ref[...])
        comb_ref[CTX_DIM:IN_FEAT, :] = emb_ref[...]

    acc = lax.dot_general(
        wt_ref[...], comb_ref[...],
        (((0,), (0,)), ((), ())),
        preferred_element_type=jnp.float32,
    )
    out_ref[...] = acc + jnp.transpose(b_ref[...])


@jax.jit
def kernel(jrnl_variable, context_vectors, emb_table, W, b):
    idx = jrnl_variable.reshape((BATCH,))
    c = idx // _LANES
    l3 = (idx % _LANES).reshape((_NGRP, _GRP, 1))
    table_t = emb_table.T
    wt = W.T
    b2d = b.reshape((1, MESH_SIZE))

    emb_t = pl.pallas_call(
        _gather_body,
        grid_spec=pltpu.PrefetchScalarGridSpec(
            num_scalar_prefetch=1,
            grid=(1,),
            in_specs=[
                pl.BlockSpec(memory_space=pl.ANY),
                pl.BlockSpec((_NGRP, _GRP, 1), lambda g, c_ref: (0, 0, 0)),
            ],
            out_specs=pl.BlockSpec((JRNL_DIM, BATCH), lambda g, c_ref: (0, 0)),
            scratch_shapes=[
                pltpu.VMEM((BATCH, JRNL_DIM, _LANES), jnp.float32),
                pltpu.SemaphoreType.DMA,
            ],
        ),
        out_shape=jax.ShapeDtypeStruct((JRNL_DIM, BATCH), jnp.float32),
        compiler_params=pltpu.CompilerParams(
            dimension_semantics=("arbitrary",),
        ),
    )(c, table_t, l3)

    n_blocks = pl.cdiv(MESH_SIZE, _BLK_N)
    out_t = pl.pallas_call(
        _mm_body,
        grid=(n_blocks,),
        in_specs=[
            pl.BlockSpec((IN_FEAT, _BLK_N), lambda i: (0, i)),
            pl.BlockSpec((BATCH, CTX_DIM), lambda i: (0, 0)),
            pl.BlockSpec((JRNL_DIM, BATCH), lambda i: (0, 0)),
            pl.BlockSpec((1, _BLK_N), lambda i: (0, i)),
        ],
        out_specs=pl.BlockSpec((_BLK_N, BATCH), lambda i: (i, 0)),
        out_shape=jax.ShapeDtypeStruct((MESH_SIZE, BATCH), jnp.float32),
        scratch_shapes=[pltpu.VMEM((IN_FEAT, BATCH), jnp.float32)],
        compiler_params=pltpu.CompilerParams(
            dimension_semantics=("arbitrary",),
        ),
    )(wt, context_vectors, emb_t, b2d)
    return out_t.T
